# baseline (speedup 1.0000x reference)
import functools

import jax
import jax.numpy as jnp
from jax import lax
from jax.experimental import pallas as pl
from jax.experimental.pallas import tpu as pltpu
from jax.experimental.pallas import tpu_sc as plsc

B = 4
S = 8192
D = 1024
L = 16
NC = 2
NS = 16
NW = NC * NS
S_PER_W = S // NW
K = 32
NCHUNK = S_PER_W // K


def _body(ids_hbm, tok_hbm, pos_hbm, out_hbm, idx_v, rows_v, pos_v, sem):
    wid = lax.axis_index("s") * NC + lax.axis_index("c")
    base = wid * S_PER_W

    def chunk(j, carry):
        off = base + j * K
        pltpu.sync_copy(pos_hbm.at[pl.ds(off, K)], pos_v)
        for b in range(B):
            pltpu.sync_copy(ids_hbm.at[b, pl.ds(off, K)], idx_v.at[b])
        for b in range(B):
            pltpu.async_copy(tok_hbm.at[idx_v.at[b]], rows_v, sem).wait()

            def add_row(r, c2):
                def add_vec(l, c3):
                    sl = pl.ds(l * L, L)
                    rows_v[r, sl] = rows_v[r, sl] + pos_v[r, sl]
                    return c3
                return lax.fori_loop(0, D // L, add_vec, c2)

            lax.fori_loop(0, K, add_row, 0)
            pltpu.sync_copy(rows_v, out_hbm.at[b, pl.ds(off, K)])
        return carry

    lax.fori_loop(0, NCHUNK, chunk, 0)


def kernel(input_ids, tok_table, pos_table):
    mesh = plsc.VectorSubcoreMesh(core_axis_name="c", subcore_axis_name="s")
    k = pl.kernel(
        _body,
        out_type=jax.ShapeDtypeStruct((B, S, D), jnp.float32),
        mesh=mesh,
        scratch_types=[
            pltpu.VMEM((B, K), jnp.int32),
            pltpu.VMEM((K, D), jnp.float32),
            pltpu.VMEM((K, D), jnp.float32),
            pltpu.SemaphoreType.DMA,
        ],
    )
    return k(input_ids, tok_table, pos_table)


# speedup vs baseline: 3.1817x; 3.1817x over previous
import jax
import jax.numpy as jnp
from jax import lax
from jax.experimental import pallas as pl
from jax.experimental.pallas import tpu as pltpu
from jax.experimental.pallas import tpu_sc as plsc

B = 4
S = 8192
D = 1024
L = 16
NC = 2
NS = 16
NW = NC * NS
S_PER_W = S // NW
K = 16
NCHUNK = S_PER_W // K
NPAIR = NCHUNK // 2


def _body(ids_hbm, tok_hbm, pos_hbm, out_hbm,
          idx_v, rows0, rows1, pos0, pos1, obuf0, obuf1,
          gsem0, gsem1, osem0, osem1, psem0, psem1, isem0, isem1):
    wid = lax.axis_index("s") * NC + lax.axis_index("c")
    base = wid * S_PER_W
    rows = (rows0, rows1)
    poss = (pos0, pos1)
    obufs = (obuf0, obuf1)
    gsems = (gsem0, gsem1)
    osems = (osem0, osem1)
    psems = (psem0, psem1)
    isems = (isem0, isem1)

    SEG = 4
    SEGV = D // L // SEG

    def add_chunk(rows_ref, pos_ref, out_ref):
        @plsc.parallel_loop(0, K * SEG, unroll=2)
        def _(i):
            r = i // SEG
            c0 = (i % SEG) * (SEGV * L)
            for l in range(SEGV):
                sl = pl.ds(c0 + l * L, L)
                out_ref[r, sl] = rows_ref[r, sl] + pos_ref[r, sl]

    def prefetch(jp, off):
        pltpu.async_copy(pos_hbm.at[pl.ds(off, K)], poss[jp], psems[jp])
        for b in range(B):
            pltpu.async_copy(ids_hbm.at[b, pl.ds(off, K)], idx_v.at[jp, b],
                             isems[jp])

    def wait_pos(jp):
        pltpu.make_async_copy(pos_hbm.at[pl.ds(0, K)], poss[jp],
                              psems[jp]).wait()

    def wait_idx(jp):
        for b in range(B):
            pltpu.make_async_copy(ids_hbm.at[0, pl.ds(0, K)],
                                  idx_v.at[jp, b], isems[jp]).wait()

    def wait_gather(p, jp, b):
        pltpu.make_async_copy(tok_hbm.at[idx_v.at[jp, b]], rows[p],
                              gsems[p]).wait()

    def wait_write(p):
        pltpu.make_async_copy(obufs[p], out_hbm.at[0, pl.ds(0, K)],
                              osems[p]).wait()

    for b in range(B):
        pltpu.sync_copy(ids_hbm.at[b, pl.ds(base, K)], idx_v.at[0, b])
    pltpu.sync_copy(pos_hbm.at[pl.ds(base, K)], pos0)
    pltpu.async_copy(tok_hbm.at[idx_v.at[0, 0]], rows0, gsem0)

    def pair(m, carry):
        for jj in range(2):
            j = 2 * m + jj
            off = base + j * K
            pos_ref = poss[jj]
            if jj == 0:
                @pl.when(m > 0)
                def _():
                    wait_pos(0)
            else:
                wait_pos(1)
            for b in range(B):
                p = b % 2
                q = 1 - p
                wait_gather(p, jj, b)
                if b < B - 1:
                    pltpu.async_copy(tok_hbm.at[idx_v.at[jj, b + 1]],
                                     rows[q], gsems[q])
                elif jj == 0:
                    wait_idx(1)
                    pltpu.async_copy(tok_hbm.at[idx_v.at[1, 0]],
                                     rows[q], gsems[q])
                else:
                    @pl.when(m < NPAIR - 1)
                    def _():
                        wait_idx(0)
                        pltpu.async_copy(tok_hbm.at[idx_v.at[0, 0]],
                                         rows[q], gsems[q])
                if b == 0:
                    if jj == 0:
                        prefetch(1, off + K)
                    else:
                        @pl.when(m < NPAIR - 1)
                        def _():
                            prefetch(0, off + K)
                if jj == 0 and b < 2:
                    @pl.when(m > 0)
                    def _():
                        wait_write(p)
                else:
                    wait_write(p)
                add_chunk(rows[p], pos_ref, obufs[p])
                pltpu.async_copy(obufs[p], out_hbm.at[b, pl.ds(off, K)],
                                 osems[p])
        return carry

    lax.fori_loop(0, NPAIR, pair, 0)
    wait_write(0)
    wait_write(1)


def kernel(input_ids, tok_table, pos_table):
    mesh = plsc.VectorSubcoreMesh(core_axis_name="c", subcore_axis_name="s")
    k = pl.kernel(
        _body,
        out_type=jax.ShapeDtypeStruct((B, S, D), jnp.float32),
        mesh=mesh,
        scratch_types=[
            pltpu.VMEM((2, B, K), jnp.int32),
            pltpu.VMEM((K, D), jnp.float32),
            pltpu.VMEM((K, D), jnp.float32),
            pltpu.VMEM((K, D), jnp.float32),
            pltpu.VMEM((K, D), jnp.float32),
            pltpu.VMEM((K, D), jnp.float32),
            pltpu.VMEM((K, D), jnp.float32),
            pltpu.SemaphoreType.DMA,
            pltpu.SemaphoreType.DMA,
            pltpu.SemaphoreType.DMA,
            pltpu.SemaphoreType.DMA,
            pltpu.SemaphoreType.DMA,
            pltpu.SemaphoreType.DMA,
            pltpu.SemaphoreType.DMA,
            pltpu.SemaphoreType.DMA,
        ],
    )
    return k(input_ids, tok_table, pos_table)


# speedup vs baseline: 3.5756x; 1.1238x over previous
import jax
import jax.numpy as jnp
from jax import lax
from jax.experimental import pallas as pl
from jax.experimental.pallas import tpu as pltpu
from jax.experimental.pallas import tpu_sc as plsc

B = 4
S = 8192
D = 1024
L = 16
NC = 2
NS = 16
NW = NC * NS
S_PER_W = S // NW
K = 8
NCHUNK = S_PER_W // K
NPAIR = NCHUNK // 2
SEG = 4
SEGV = D // L // SEG


def _body(ids_hbm, tok_hbm, pos_hbm, out_hbm,
          idx_v, b00, b01, b02, b03, b10, b11, b12, b13, pos0, pos1,
          gsem0, gsem1, osem0, osem1, psem0, psem1, isem0, isem1):
    wid = lax.axis_index("s") * NC + lax.axis_index("c")
    base = wid * S_PER_W
    bufs = ((b00, b01, b02, b03), (b10, b11, b12, b13))
    poss = (pos0, pos1)
    gsems = (gsem0, gsem1)
    osems = (osem0, osem1)
    psems = (psem0, psem1)
    isems = (isem0, isem1)

    def add_chunk(jp):
        bb = bufs[jp]
        pos_ref = poss[jp]

        @plsc.parallel_loop(0, K * SEG)
        def _(i):
            r = i // SEG
            c0 = (i % SEG) * (SEGV * L)
            for l in range(SEGV):
                sl = pl.ds(c0 + l * L, L)
                p = pos_ref[r, sl]
                for b in range(B):
                    plsc.addupdate(bb[b].at[r, sl], p)

    def issue_gathers(jp, ip):
        for b in range(B):
            pltpu.async_copy(tok_hbm.at[idx_v.at[ip, b]], bufs[jp][b],
                             gsems[jp])

    def wait_gathers(jp):
        for b in range(B):
            pltpu.make_async_copy(tok_hbm.at[idx_v.at[0, 0]], bufs[jp][b],
                                  gsems[jp]).wait()

    def issue_writes(jp, off):
        for b in range(B):
            pltpu.async_copy(bufs[jp][b], out_hbm.at[b, pl.ds(off, K)],
                             osems[jp])

    def wait_writes(jp):
        for b in range(B):
            pltpu.make_async_copy(bufs[jp][b], out_hbm.at[0, pl.ds(0, K)],
                                  osems[jp]).wait()

    def prefetch(jp, off):
        pltpu.async_copy(pos_hbm.at[pl.ds(off, K)], poss[jp], psems[jp])
        for b in range(B):
            pltpu.async_copy(ids_hbm.at[b, pl.ds(off, K)], idx_v.at[jp, b],
                             isems[jp])

    def wait_pos(jp):
        pltpu.make_async_copy(pos_hbm.at[pl.ds(0, K)], poss[jp],
                              psems[jp]).wait()

    def wait_idx(jp):
        for b in range(B):
            pltpu.make_async_copy(ids_hbm.at[0, pl.ds(0, K)],
                                  idx_v.at[jp, b], isems[jp]).wait()

    for b in range(B):
        pltpu.sync_copy(ids_hbm.at[b, pl.ds(base, K)], idx_v.at[0, b])
    pltpu.sync_copy(pos_hbm.at[pl.ds(base, K)], pos0)
    prefetch(1, base + K)
    issue_gathers(0, 0)

    def pair(m, carry):
        for jj in range(2):
            j = 2 * m + jj
            off = base + j * K
            wait_gathers(jj)
            if jj == 0:
                @pl.when(m > 0)
                def _():
                    wait_pos(0)
            else:
                wait_pos(1)
            if jj == 0:
                @pl.when(m > 0)
                def _():
                    wait_writes(1)
                wait_idx(1)
                issue_gathers(1, 1)
            else:
                wait_writes(0)

                @pl.when(m < NPAIR - 1)
                def _():
                    wait_idx(0)
                    issue_gathers(0, 0)
            add_chunk(jj)
            issue_writes(jj, off)
            @pl.when(m < NPAIR - 1)
            def _():
                prefetch(jj, off + 2 * K)
        return carry

    lax.fori_loop(0, NPAIR, pair, 0)
    wait_writes(1)


def kernel(input_ids, tok_table, pos_table):
    mesh = plsc.VectorSubcoreMesh(core_axis_name="c", subcore_axis_name="s")
    k = pl.kernel(
        _body,
        out_type=jax.ShapeDtypeStruct((B, S, D), jnp.float32),
        mesh=mesh,
        scratch_types=[
            pltpu.VMEM((2, B, K), jnp.int32),
            pltpu.VMEM((K, D), jnp.float32),
            pltpu.VMEM((K, D), jnp.float32),
            pltpu.VMEM((K, D), jnp.float32),
            pltpu.VMEM((K, D), jnp.float32),
            pltpu.VMEM((K, D), jnp.float32),
            pltpu.VMEM((K, D), jnp.float32),
            pltpu.VMEM((K, D), jnp.float32),
            pltpu.VMEM((K, D), jnp.float32),
            pltpu.VMEM((K, D), jnp.float32),
            pltpu.VMEM((K, D), jnp.float32),
            pltpu.SemaphoreType.DMA,
            pltpu.SemaphoreType.DMA,
            pltpu.SemaphoreType.DMA,
            pltpu.SemaphoreType.DMA,
            pltpu.SemaphoreType.DMA,
            pltpu.SemaphoreType.DMA,
            pltpu.SemaphoreType.DMA,
            pltpu.SemaphoreType.DMA,
        ],
    )
    return k(input_ids, tok_table, pos_table)
